# baseline (speedup 1.0000x reference)
import jax
import jax.numpy as jnp
import numpy as np
from jax.experimental import pallas as pl
from jax.experimental.pallas import tpu as pltpu

N = 10000
IN_F = 256
OUT_F = 256
JOINT = 3 * OUT_F

_PERM = np.concatenate([np.arange(k, JOINT, 3) for k in range(3)])

_BM_A = 1000
_BM_B = 200


def _support_body(x_ref, w1_ref, b1_ref, w2_ref, b2_ref, out_ref):
    x = x_ref[...].astype(jnp.bfloat16)
    a = jnp.dot(x, w1_ref[...], preferred_element_type=jnp.float32) + b1_ref[...]
    b = jnp.dot(x, w2_ref[...], preferred_element_type=jnp.float32) + b2_ref[...]
    s = a * b
    iq = s[:, :OUT_F] + s[:, OUT_F:2 * OUT_F] + s[:, 2 * OUT_F:]
    iq = jnp.sign(iq) * jnp.sqrt(jnp.abs(iq))
    norm = jnp.maximum(jnp.sqrt(jnp.sum(iq * iq, axis=1, keepdims=True)), 1e-12)
    out_ref[...] = (iq / norm).astype(jnp.bfloat16)


def _adj_body(adj_ref, sup_ref, bias_ref, out_ref):
    a = adj_ref[...].astype(jnp.bfloat16)
    out_ref[...] = (
        jnp.dot(a, sup_ref[...], preferred_element_type=jnp.float32)
        + bias_ref[...]
    )


def kernel(input, adj, W1, b1, W2, b2, bias):
    w1p = W1[:, _PERM].astype(jnp.bfloat16)
    w2p = W2[:, _PERM].astype(jnp.bfloat16)
    b1p = b1[_PERM].reshape(1, JOINT)
    b2p = b2[_PERM].reshape(1, JOINT)

    support = pl.pallas_call(
        _support_body,
        grid=(N // _BM_A,),
        in_specs=[
            pl.BlockSpec((_BM_A, IN_F), lambda i: (i, 0)),
            pl.BlockSpec((IN_F, JOINT), lambda i: (0, 0)),
            pl.BlockSpec((1, JOINT), lambda i: (0, 0)),
            pl.BlockSpec((IN_F, JOINT), lambda i: (0, 0)),
            pl.BlockSpec((1, JOINT), lambda i: (0, 0)),
        ],
        out_specs=pl.BlockSpec((_BM_A, OUT_F), lambda i: (i, 0)),
        out_shape=jax.ShapeDtypeStruct((N, OUT_F), jnp.bfloat16),
        compiler_params=pltpu.CompilerParams(
            dimension_semantics=("parallel",),
        ),
    )(input, w1p, b1p, w2p, b2p)

    out = pl.pallas_call(
        _adj_body,
        grid=(N // _BM_B,),
        in_specs=[
            pl.BlockSpec((_BM_B, N), lambda i: (i, 0)),
            pl.BlockSpec((N, OUT_F), lambda i: (0, 0)),
            pl.BlockSpec((1, OUT_F), lambda i: (0, 0)),
        ],
        out_specs=pl.BlockSpec((_BM_B, OUT_F), lambda i: (i, 0)),
        out_shape=jax.ShapeDtypeStruct((N, OUT_F), jnp.float32),
        compiler_params=pltpu.CompilerParams(
            dimension_semantics=("parallel",),
        ),
    )(adj, support, bias.reshape(1, OUT_F))
    return out


# speedup vs baseline: 1.0191x; 1.0191x over previous
import jax
import jax.numpy as jnp
import numpy as np
from jax.experimental import pallas as pl
from jax.experimental.pallas import tpu as pltpu

N = 10000
IN_F = 256
OUT_F = 256
JOINT = 3 * OUT_F

_PERM = np.concatenate([np.arange(k, JOINT, 3) for k in range(3)])

_BM_A = 1000
_BM_B = 400


def _fused_body(x_ref, w1_ref, b1_ref, w2_ref, b2_ref, adj_ref, bias_ref,
                out_ref, sup_ref):
    i = pl.program_id(0)

    @pl.when(i == 0)
    def _():
        def chunk(c, carry):
            x = x_ref[pl.ds(c * _BM_A, _BM_A), :].astype(jnp.bfloat16)
            a = jnp.dot(x, w1_ref[...], preferred_element_type=jnp.float32) + b1_ref[...]
            b = jnp.dot(x, w2_ref[...], preferred_element_type=jnp.float32) + b2_ref[...]
            s = a * b
            iq = s[:, :OUT_F] + s[:, OUT_F:2 * OUT_F] + s[:, 2 * OUT_F:]
            iq = jnp.sign(iq) * jnp.sqrt(jnp.abs(iq))
            nrm = jnp.maximum(
                jnp.sqrt(jnp.sum(iq * iq, axis=1, keepdims=True)), 1e-12)
            sup_ref[pl.ds(c * _BM_A, _BM_A), :] = iq / nrm
            return carry
        jax.lax.fori_loop(0, N // _BM_A, chunk, 0)

    out_ref[...] = (
        jax.lax.dot_general(
            adj_ref[...], sup_ref[...],
            dimension_numbers=(((1,), (0,)), ((), ())),
            precision=jax.lax.Precision.DEFAULT,
            preferred_element_type=jnp.float32,
        )
        + bias_ref[...]
    )


def kernel(input, adj, W1, b1, W2, b2, bias):
    w1p = W1[:, _PERM].astype(jnp.bfloat16)
    w2p = W2[:, _PERM].astype(jnp.bfloat16)
    b1p = b1[_PERM].reshape(1, JOINT)
    b2p = b2[_PERM].reshape(1, JOINT)

    return pl.pallas_call(
        _fused_body,
        grid=(N // _BM_B,),
        in_specs=[
            pl.BlockSpec((N, IN_F), lambda i: (0, 0)),
            pl.BlockSpec((IN_F, JOINT), lambda i: (0, 0)),
            pl.BlockSpec((1, JOINT), lambda i: (0, 0)),
            pl.BlockSpec((IN_F, JOINT), lambda i: (0, 0)),
            pl.BlockSpec((1, JOINT), lambda i: (0, 0)),
            pl.BlockSpec((_BM_B, N), lambda i: (i, 0)),
            pl.BlockSpec((1, OUT_F), lambda i: (0, 0)),
        ],
        out_specs=pl.BlockSpec((_BM_B, OUT_F), lambda i: (i, 0)),
        out_shape=jax.ShapeDtypeStruct((N, OUT_F), jnp.float32),
        scratch_shapes=[pltpu.VMEM((N, OUT_F), jnp.float32)],
        compiler_params=pltpu.CompilerParams(
            dimension_semantics=("arbitrary",),
        ),
    )(input, w1p, b1p, w2p, b2p, adj, bias.reshape(1, OUT_F))


# speedup vs baseline: 1.0460x; 1.0265x over previous
import jax
import jax.numpy as jnp
import numpy as np
from jax.experimental import pallas as pl
from jax.experimental.pallas import tpu as pltpu

N = 10000
IN_F = 256
OUT_F = 256
JOINT = 3 * OUT_F

_PERM = np.concatenate([np.arange(k, JOINT, 3) for k in range(3)])

_BM_A = 1000
_BM_B = 400


def _fused_body(x_ref, w1_ref, b1_ref, w2_ref, b2_ref, adj_ref, bias_ref,
                out_ref, sup_ref):
    i = pl.program_id(0)

    @pl.when(i == 0)
    def _():
        def chunk(c, carry):
            x = x_ref[pl.ds(c * _BM_A, _BM_A), :].astype(jnp.bfloat16)
            a = jnp.dot(x, w1_ref[...], preferred_element_type=jnp.float32) + b1_ref[...]
            b = jnp.dot(x, w2_ref[...], preferred_element_type=jnp.float32) + b2_ref[...]
            s = a * b
            iq = s[:, :OUT_F] + s[:, OUT_F:2 * OUT_F] + s[:, 2 * OUT_F:]
            absiq = jnp.abs(iq)
            ssq = jnp.sum(absiq, axis=1, keepdims=True)
            sgn_sqrt = iq * jax.lax.rsqrt(jnp.maximum(absiq, 1e-30))
            rnorm = jax.lax.rsqrt(jnp.maximum(ssq, 1e-24))
            sup_ref[pl.ds(c * _BM_A, _BM_A), :] = sgn_sqrt * rnorm
            return carry
        jax.lax.fori_loop(0, N // _BM_A, chunk, 0)

    out_ref[...] = (
        jax.lax.dot_general(
            adj_ref[...], sup_ref[...],
            dimension_numbers=(((1,), (0,)), ((), ())),
            precision=jax.lax.Precision.DEFAULT,
            preferred_element_type=jnp.float32,
        )
        + bias_ref[...]
    )


def kernel(input, adj, W1, b1, W2, b2, bias):
    w1p = W1[:, _PERM].astype(jnp.bfloat16)
    w2p = W2[:, _PERM].astype(jnp.bfloat16)
    b1p = b1[_PERM].reshape(1, JOINT)
    b2p = b2[_PERM].reshape(1, JOINT)

    return pl.pallas_call(
        _fused_body,
        grid=(N // _BM_B,),
        in_specs=[
            pl.BlockSpec((N, IN_F), lambda i: (0, 0)),
            pl.BlockSpec((IN_F, JOINT), lambda i: (0, 0)),
            pl.BlockSpec((1, JOINT), lambda i: (0, 0)),
            pl.BlockSpec((IN_F, JOINT), lambda i: (0, 0)),
            pl.BlockSpec((1, JOINT), lambda i: (0, 0)),
            pl.BlockSpec((_BM_B, N), lambda i: (i, 0)),
            pl.BlockSpec((1, OUT_F), lambda i: (0, 0)),
        ],
        out_specs=pl.BlockSpec((_BM_B, OUT_F), lambda i: (i, 0)),
        out_shape=jax.ShapeDtypeStruct((N, OUT_F), jnp.float32),
        scratch_shapes=[pltpu.VMEM((N, OUT_F), jnp.float32)],
        compiler_params=pltpu.CompilerParams(
            dimension_semantics=("arbitrary",),
        ),
    )(input, w1p, b1p, w2p, b2p, adj, bias.reshape(1, OUT_F))
